# baseline (speedup 1.0000x reference)
import jax
import jax.numpy as jnp
from jax import lax
from jax.experimental import pallas as pl
from jax.experimental.pallas import tpu as pltpu
from jax.experimental.pallas import tpu_sc as plsc

_B = 16384
_V = 1000
_NC = 2
_NS = 16
_L = 16
_NW = _NC * _NS
_ROWS_PER_W = _B // _NW
_R = 32
_CH = _ROWS_PER_W // _R
_BUF = _R * _V


def _onehot_body(x_hbm, out_hbm, x_v, buf0, buf1, sem0, sem1):
    wid = lax.axis_index("s") * _NC + lax.axis_index("c")
    base = wid * _ROWS_PER_W

    pltpu.sync_copy(x_hbm.at[pl.ds(base * 1, _ROWS_PER_W)], x_v)

    bufs = (buf0, buf1)
    sems = (sem0, sem1)

    def _zf(i, carry):
        buf0[pl.ds(i * _L, _L)] = jnp.zeros((_L,), jnp.float32)
        buf1[pl.ds(i * _L, _L)] = jnp.zeros((_L,), jnp.float32)
        return carry

    lax.fori_loop(0, _BUF // _L, _zf, 0)

    ones = jnp.ones((_L,), jnp.float32)
    zeros = jnp.zeros((_L,), jnp.float32)
    lane = lax.iota(jnp.int32, _L)

    def _scatter(buf, ch, val):
        for j in range(_R // _L):
            rloc = lane + (j * _L)
            xv = x_v[pl.ds(ch * _R + j * _L, _L)]
            plsc.store_scatter(buf, [rloc * _V + xv], val)

    handles = [None, None]
    for ch in range(_CH):
        b = ch % 2
        if handles[b] is not None:
            handles[b].wait()
            _scatter(bufs[b], ch - 2, zeros)
        _scatter(bufs[b], ch, ones)
        handles[b] = pltpu.async_copy(
            bufs[b], out_hbm.at[pl.ds((base + ch * _R) * _V, _BUF)], sems[b]
        )
    handles[_CH % 2].wait()
    handles[(_CH + 1) % 2].wait()


def kernel(x):
    mesh = plsc.VectorSubcoreMesh(core_axis_name="c", subcore_axis_name="s")
    run = pl.kernel(
        _onehot_body,
        out_type=jax.ShapeDtypeStruct((_B * _V,), jnp.float32),
        mesh=mesh,
        compiler_params=pltpu.CompilerParams(needs_layout_passes=False),
        scratch_types=[
            pltpu.VMEM((_ROWS_PER_W,), jnp.int32),
            pltpu.VMEM((_BUF,), jnp.float32),
            pltpu.VMEM((_BUF,), jnp.float32),
            pltpu.SemaphoreType.DMA,
            pltpu.SemaphoreType.DMA,
        ],
    )
    return run(x.astype(jnp.int32)).reshape(_B, _V)


# speedup vs baseline: 1.6609x; 1.6609x over previous
import jax
import jax.numpy as jnp
from jax import lax
from jax.experimental import pallas as pl
from jax.experimental.pallas import tpu as pltpu
from jax.experimental.pallas import tpu_sc as plsc

_B = 16384
_V = 1000
_NC = 2
_NS = 16
_L = 16
_NW = _NC * _NS
_ROWS_PER_W = _B // _NW
_R = 32
_CH = _ROWS_PER_W // _R


def _onehot_body(x_hbm, out_hbm, x_v, buf0, buf1, sem0, sem1):
    wid = lax.axis_index("s") * _NC + lax.axis_index("c")
    base = wid * _ROWS_PER_W

    pltpu.sync_copy(x_hbm.at[pl.ds(base * 1, _ROWS_PER_W)], x_v)

    bufs = (buf0, buf1)
    sems = (sem0, sem1)

    zrow = jnp.zeros((_L,), jnp.float32)

    @pl.loop(0, _R)
    def _zf(r):
        for buf in bufs:
            for c in range(0, _V - _L + 1, _L):
                buf[r, pl.ds(c, _L)] = zrow
            buf[r, pl.ds(_V - _L, _L)] = zrow

    ones = jnp.ones((_L,), jnp.float32)
    zeros = jnp.zeros((_L,), jnp.float32)
    lane = lax.iota(jnp.int32, _L)

    def _scatter(buf, ch, val):
        for j in range(_R // _L):
            rloc = lane + (j * _L)
            xv = x_v[pl.ds(ch * _R + j * _L, _L)]
            plsc.store_scatter(buf, [rloc, xv], val)

    handles = [None, None]
    for ch in range(_CH):
        b = ch % 2
        if handles[b] is not None:
            handles[b].wait()
            _scatter(bufs[b], ch - 2, zeros)
        _scatter(bufs[b], ch, ones)
        handles[b] = pltpu.async_copy(
            bufs[b], out_hbm.at[pl.ds(base + ch * _R, _R)], sems[b]
        )
    handles[_CH % 2].wait()
    handles[(_CH + 1) % 2].wait()


def kernel(x):
    mesh = plsc.VectorSubcoreMesh(core_axis_name="c", subcore_axis_name="s")
    run = pl.kernel(
        _onehot_body,
        out_type=jax.ShapeDtypeStruct((_B, _V), jnp.float32),
        mesh=mesh,
        compiler_params=pltpu.CompilerParams(needs_layout_passes=False),
        scratch_types=[
            pltpu.VMEM((_ROWS_PER_W,), jnp.int32),
            pltpu.VMEM((_R, _V), jnp.float32),
            pltpu.VMEM((_R, _V), jnp.float32),
            pltpu.SemaphoreType.DMA,
            pltpu.SemaphoreType.DMA,
        ],
    )
    return run(x.astype(jnp.int32))
